# baseline (speedup 1.0000x reference)
import functools

import jax
import jax.numpy as jnp
from jax import lax
from jax.experimental import pallas as pl
from jax.experimental.pallas import tpu as pltpu
from jax.experimental.pallas import tpu_sc as plsc

_NUM_OBJS = 151
_NUM_RELS = 51
_ROW_PAD = 64
_BATCH = 16384

_NUM_CORES = 2
_NUM_SUBCORES = 16
_NW = _NUM_CORES * _NUM_SUBCORES
_B_PER_W = _BATCH // _NW
_LANES = 16


def _build_sc_kernel():
    mesh = plsc.VectorSubcoreMesh(core_axis_name="c", subcore_axis_name="s")

    @functools.partial(
        pl.kernel,
        mesh=mesh,
        out_type=jax.ShapeDtypeStruct((_BATCH, _ROW_PAD), jnp.float32),
        compiler_params=pltpu.CompilerParams(use_tc_tiling_on_sc=False),
        scratch_types=[
            pltpu.VMEM((_B_PER_W,), jnp.int32),
            pltpu.VMEM((_B_PER_W,), jnp.int32),
            pltpu.VMEM((_B_PER_W,), jnp.int32),
            pltpu.VMEM((_B_PER_W, _ROW_PAD), jnp.float32),
            pltpu.SemaphoreType.DMA,
        ],
    )
    def sc_kernel(l0_hbm, l1_hbm, w_hbm, out_hbm, a_v, b_v, idx_v, rows_v, sem):
        wid = lax.axis_index("s") * _NUM_CORES + lax.axis_index("c")
        base = wid * _B_PER_W
        pltpu.sync_copy(l0_hbm.at[pl.ds(base, _B_PER_W)], a_v)
        pltpu.sync_copy(l1_hbm.at[pl.ds(base, _B_PER_W)], b_v)
        for j in range(_B_PER_W // _LANES):
            s = pl.ds(j * _LANES, _LANES)
            idx_v[s] = a_v[s] * _NUM_OBJS + b_v[s]
        pltpu.async_copy(w_hbm.at[idx_v], rows_v, sem).wait()
        pltpu.sync_copy(rows_v, out_hbm.at[pl.ds(base, _B_PER_W)])

    return sc_kernel


_SC_KERNEL = _build_sc_kernel()


@jax.jit
def kernel(labels, W):
    l0 = labels[:, 0]
    l1 = labels[:, 1]
    w_pad = jnp.pad(W, ((0, 0), (0, _ROW_PAD - _NUM_RELS)))
    out_pad = _SC_KERNEL(l0, l1, w_pad)
    return out_pad[:, :_NUM_RELS]
